# baseline (speedup 1.0000x reference)
import functools

import jax
import jax.numpy as jnp
from jax import lax
from jax.experimental import pallas as pl
from jax.experimental.pallas import tpu as pltpu
from jax.experimental.pallas import tpu_sc as plsc

_N = 10000
_E = 320000
_D = 128
_NC = 2
_NS = 16
_NW = _NC * _NS
_EPT = _E // _NW
_B = 80
_CH = _EPT // _B
_BLK = 25
_NBLK = _CH // _BLK
_NB = 4
_LG = 4
_NP = 10112
_RPT = _NP // _NS
_NPD = 10240
_RPD = _NPD // _NS


def _make_seg_sum(compute_deg):
    mesh = plsc.VectorSubcoreMesh(
        core_axis_name="c", subcore_axis_name="s",
        num_cores=_NC, num_subcores=_NS)

    out_type = [jax.ShapeDtypeStruct((_NC, _NP, _D), jnp.float32)]
    if compute_deg:
        out_type.append(jax.ShapeDtypeStruct((_NC * _NPD,), jnp.float32))

    @functools.partial(
        pl.kernel,
        out_type=tuple(out_type),
        mesh=mesh,
        scratch_types=[
            pltpu.VMEM((_BLK, _B), jnp.int32),
            pltpu.VMEM((_BLK, _B), jnp.int32),
            pltpu.VMEM((_NB, _B, _D), jnp.float32),
            pltpu.VMEM((_B,), jnp.float32),
            pltpu.VMEM_SHARED((_NP, _D), jnp.float32),
            pltpu.VMEM_SHARED((_NPD,), jnp.float32),
            pltpu.SemaphoreType.DMA,
            pltpu.SemaphoreType.DMA,
            pltpu.SemaphoreType.DMA,
            pltpu.SemaphoreType.DMA,
            pltpu.SemaphoreType.DMA,
        ],
    )
    def seg_sum(table_hbm, edges_hbm, zeros2_hbm, zeros1_hbm, ones_hbm,
                *refs):
        if compute_deg:
            out_hbm, deg_hbm = refs[0], refs[1]
            rest = refs[2:]
        else:
            out_hbm = refs[0]
            rest = refs[1:]
        (src_v, dst_v, rows_v, ones_v, acc_sh, deg_sh,
         gsem, sA, sB, dA, dB) = rest

        c = lax.axis_index("c")
        s = lax.axis_index("s")
        wid = s * _NC + c
        r0 = s * _RPT
        pltpu.sync_copy(zeros2_hbm, acc_sh.at[pl.ds(r0, _RPT)])
        if compute_deg:
            pltpu.sync_copy(zeros1_hbm, deg_sh.at[pl.ds(s * _RPD, _RPD)])
            pltpu.sync_copy(ones_hbm, ones_v)
        plsc.subcore_barrier()

        def fire_gather(j):
            pltpu.async_copy(table_hbm.at[src_v.at[j]],
                             rows_v.at[lax.rem(j, _NB)], gsem)

        def drain_gather():
            pltpu.make_async_copy(
                zeros2_hbm.at[pl.ds(0, _B)], rows_v.at[0], gsem).wait()

        def fire_scatter(j, rsem, degsem):
            d = pltpu.async_copy(rows_v.at[lax.rem(j, _NB)],
                                 acc_sh.at[dst_v.at[j]], rsem, add=True)
            e = None
            if compute_deg:
                e = pltpu.async_copy(ones_v, deg_sh.at[dst_v.at[j]], degsem,
                                     add=True)
            return d, e

        def wait_scatter(de):
            de[0].wait()
            if de[1] is not None:
                de[1].wait()

        _NPAIR = (_BLK - _LG) // 2

        def block(k, carry):
            pltpu.sync_copy(edges_hbm.at[0, wid, k], src_v)
            pltpu.sync_copy(edges_hbm.at[1, wid, k], dst_v)
            for j in range(_LG):
                fire_gather(j)

            def body(p, c2):
                a = 2 * p
                b = a + 1
                drain_gather()
                da = fire_scatter(a, sA, dA)
                drain_gather()
                db = fire_scatter(b, sB, dB)
                wait_scatter(da)
                fire_gather(a + _LG)
                wait_scatter(db)
                fire_gather(b + _LG)
                return c2

            lax.fori_loop(0, _NPAIR, body, 0)
            for j in range(2 * _NPAIR, _BLK):
                drain_gather()
                wait_scatter(fire_scatter(j, sA, dA))
                if j + _LG < _BLK:
                    fire_gather(j + _LG)
            return carry

        lax.fori_loop(0, _NBLK, block, 0)
        plsc.subcore_barrier()
        pltpu.sync_copy(acc_sh.at[pl.ds(r0, _RPT)],
                        out_hbm.at[c, pl.ds(r0, _RPT)])
        if compute_deg:
            pltpu.sync_copy(deg_sh.at[pl.ds(s * _RPD, _RPD)],
                            deg_hbm.at[pl.ds(c * _NPD + s * _RPD, _RPD)])

    return seg_sum


_seg_sum_l1 = _make_seg_sum(True)
_seg_sum_l2 = _make_seg_sum(False)


def _bn_relu_tail(h, g, b):
    m = jnp.mean(h, axis=0, keepdims=True)
    d = h - m
    v = jnp.mean(d * d, axis=0, keepdims=True)
    return jnp.maximum(g * d * lax.rsqrt(v + 1e-5) + b, 0.0)


def _tc_body(p_ref, x_ref, deg0_ref, deg1_ref, wl_ref, bl_ref,
             wr_ref, g_ref, b_ref, h_ref):
    deg = deg0_ref[...] + deg1_ref[...]
    dinv = 1.0 / jnp.maximum(deg, 1.0)
    agg = (p_ref[0, :_N] + p_ref[1, :_N]) * dinv
    h = (jnp.dot(agg, wl_ref[...], preferred_element_type=jnp.float32)
         + bl_ref[...]
         + jnp.dot(x_ref[...], wr_ref[...], preferred_element_type=jnp.float32))
    h_ref[...] = _bn_relu_tail(h, g_ref[...], b_ref[...])


def _tc_layer(p, x, deg0, deg1, Wl, bl, Wr, g, b):
    return pl.pallas_call(
        _tc_body,
        out_shape=jax.ShapeDtypeStruct((_N, _D), jnp.float32),
    )(p, x, deg0, deg1, Wl, bl.reshape(1, _D), Wr,
      g.reshape(1, _D), b.reshape(1, _D))


def kernel(x, edge_index, Wl0, bl0, Wr0, g0, b0, Wl1, bl1, Wr1, g1, b1):
    edges = edge_index.reshape(2, _NW, _NBLK, _BLK, _B)
    zeros2 = jnp.zeros((_RPT, _D), jnp.float32)
    zeros1 = jnp.zeros((_RPD,), jnp.float32)
    ones = jnp.ones((_B,), jnp.float32)

    part1, deg = _seg_sum_l1(x, edges, zeros2, zeros1, ones)
    deg = deg.reshape(_NC, _NPD)
    deg0 = deg[0, :_N].reshape(_N, 1)
    deg1 = deg[1, :_N].reshape(_N, 1)

    h1 = _tc_layer(part1, x, deg0, deg1, Wl0, bl0, Wr0, g0, b0)

    (part2,) = _seg_sum_l2(h1, edges, zeros2, zeros1, ones)

    h2 = _tc_layer(part2, h1, deg0, deg1, Wl1, bl1, Wr1, g1, b1)

    return h2


# speedup vs baseline: 1.0210x; 1.0210x over previous
import functools

import jax
import jax.numpy as jnp
from jax import lax
from jax.experimental import pallas as pl
from jax.experimental.pallas import tpu as pltpu
from jax.experimental.pallas import tpu_sc as plsc

_N = 10000
_E = 320000
_D = 128
_NC = 2
_NS = 16
_NW = _NC * _NS
_EPT = _E // _NW
_B = 80
_CH = _EPT // _B
_BLK = 25
_NBLK = _CH // _BLK
_NB = 4
_LG = 4
_NP = 10112
_RPT = _NP // _NS
_NPD = 10240
_RPD = _NPD // _NS


def _make_seg_sum(compute_deg):
    mesh = plsc.VectorSubcoreMesh(
        core_axis_name="c", subcore_axis_name="s",
        num_cores=_NC, num_subcores=_NS)

    out_type = [jax.ShapeDtypeStruct((_NC, _NP, _D), jnp.float32)]
    if compute_deg:
        out_type.append(jax.ShapeDtypeStruct((_NC * _NPD,), jnp.float32))

    @functools.partial(
        pl.kernel,
        out_type=tuple(out_type),
        mesh=mesh,
        scratch_types=[
            pltpu.VMEM((_BLK, _B), jnp.int32),
            pltpu.VMEM((_BLK, _B), jnp.int32),
            pltpu.VMEM((_NB, _B, _D), jnp.float32),
            pltpu.VMEM((_B,), jnp.float32),
            pltpu.VMEM_SHARED((_NP, _D), jnp.float32),
            pltpu.VMEM_SHARED((_NPD,), jnp.float32),
            pltpu.SemaphoreType.DMA,
            pltpu.SemaphoreType.DMA,
        ],
    )
    def seg_sum(table_hbm, edges_hbm, zeros2_hbm, zeros1_hbm, ones_hbm,
                *refs):
        if compute_deg:
            out_hbm, deg_hbm = refs[0], refs[1]
            rest = refs[2:]
        else:
            out_hbm = refs[0]
            rest = refs[1:]
        src_v, dst_v, rows_v, ones_v, acc_sh, deg_sh, gsem, dA = rest

        c = lax.axis_index("c")
        s = lax.axis_index("s")
        wid = s * _NC + c
        r0 = s * _RPT
        pltpu.sync_copy(zeros2_hbm, rows_v.at[0])
        for i in range(7):
            pltpu.sync_copy(rows_v.at[0], acc_sh.at[pl.ds(r0 + i * _B, _B)])
        pltpu.sync_copy(rows_v.at[0, pl.ds(0, _RPT - 7 * _B)],
                        acc_sh.at[pl.ds(r0 + 7 * _B, _RPT - 7 * _B)])
        if compute_deg:
            pltpu.sync_copy(zeros1_hbm, deg_sh.at[pl.ds(s * _RPD, _RPD)])
            pltpu.sync_copy(ones_hbm, ones_v)
        plsc.subcore_barrier()

        def fire_gather(j):
            pltpu.async_copy(table_hbm.at[src_v.at[j]],
                             rows_v.at[lax.rem(j, _NB)], gsem)

        def drain_gather():
            pltpu.make_async_copy(
                zeros2_hbm.at[pl.ds(0, _B)], rows_v.at[0], gsem).wait()

        def do_scatter(j):
            if compute_deg:
                d = pltpu.async_copy(ones_v, deg_sh.at[dst_v.at[j]], dA,
                                     add=True)
                pltpu.sync_copy(rows_v.at[lax.rem(j, _NB)],
                                acc_sh.at[dst_v.at[j]], add=True)
                d.wait()
            else:
                pltpu.sync_copy(rows_v.at[lax.rem(j, _NB)],
                                acc_sh.at[dst_v.at[j]], add=True)

        def block(k, carry):
            pltpu.sync_copy(edges_hbm.at[0, wid, k], src_v)
            pltpu.sync_copy(edges_hbm.at[1, wid, k], dst_v)
            for j in range(_LG):
                fire_gather(j)

            def body(j, c2):
                drain_gather()
                do_scatter(j)
                fire_gather(j + _LG)
                return c2

            lax.fori_loop(0, _BLK - _LG, body, 0)
            for j in range(_BLK - _LG, _BLK):
                drain_gather()
                do_scatter(j)
            return carry

        lax.fori_loop(0, _NBLK, block, 0)
        plsc.subcore_barrier()
        pltpu.sync_copy(acc_sh.at[pl.ds(r0, _RPT)],
                        out_hbm.at[c, pl.ds(r0, _RPT)])
        if compute_deg:
            pltpu.sync_copy(deg_sh.at[pl.ds(s * _RPD, _RPD)],
                            deg_hbm.at[pl.ds(c * _NPD + s * _RPD, _RPD)])

    return seg_sum


_seg_sum_l1 = _make_seg_sum(True)
_seg_sum_l2 = _make_seg_sum(False)


def _bn_relu_tail(h, g, b):
    m = jnp.mean(h, axis=0, keepdims=True)
    d = h - m
    v = jnp.mean(d * d, axis=0, keepdims=True)
    return jnp.maximum(g * d * lax.rsqrt(v + 1e-5) + b, 0.0)


def _tc_body(p_ref, x_ref, deg0_ref, deg1_ref, wl_ref, bl_ref,
             wr_ref, g_ref, b_ref, h_ref):
    deg = deg0_ref[...] + deg1_ref[...]
    dinv = 1.0 / jnp.maximum(deg, 1.0)
    agg = (p_ref[0, :_N] + p_ref[1, :_N]) * dinv
    h = (jnp.dot(agg, wl_ref[...], preferred_element_type=jnp.float32)
         + bl_ref[...]
         + jnp.dot(x_ref[...], wr_ref[...], preferred_element_type=jnp.float32))
    h_ref[...] = _bn_relu_tail(h, g_ref[...], b_ref[...])


def _tc_layer(p, x, deg0, deg1, Wl, bl, Wr, g, b):
    return pl.pallas_call(
        _tc_body,
        out_shape=jax.ShapeDtypeStruct((_N, _D), jnp.float32),
    )(p, x, deg0, deg1, Wl, bl.reshape(1, _D), Wr,
      g.reshape(1, _D), b.reshape(1, _D))


def kernel(x, edge_index, Wl0, bl0, Wr0, g0, b0, Wl1, bl1, Wr1, g1, b1):
    edges = edge_index.reshape(2, _NW, _NBLK, _BLK, _B)
    zeros2 = jnp.zeros((_B, _D), jnp.float32)
    zeros1 = jnp.zeros((_RPD,), jnp.float32)
    ones = jnp.ones((_B,), jnp.float32)

    part1, deg = _seg_sum_l1(x, edges, zeros2, zeros1, ones)
    deg = deg.reshape(_NC, _NPD)
    deg0 = deg[0, :_N].reshape(_N, 1)
    deg1 = deg[1, :_N].reshape(_N, 1)

    h1 = _tc_layer(part1, x, deg0, deg1, Wl0, bl0, Wr0, g0, b0)

    (part2,) = _seg_sum_l2(h1, edges, zeros2, zeros1, ones)

    h2 = _tc_layer(part2, h1, deg0, deg1, Wl1, bl1, Wr1, g1, b1)

    return h2


# speedup vs baseline: 1.0752x; 1.0530x over previous
import functools

import jax
import jax.numpy as jnp
from jax import lax
from jax.experimental import pallas as pl
from jax.experimental.pallas import tpu as pltpu
from jax.experimental.pallas import tpu_sc as plsc

_N = 10000
_E = 320000
_D = 128
_NC = 2
_NS = 16
_NW = _NC * _NS
_EPT = _E // _NW
_B = 80
_CH = _EPT // _B
_BLK = 25
_NBLK = _CH // _BLK
_NB = 4
_LG = 4
_NP = 10112
_RPT = _NP // _NS
_NPD = 10240
_RPD = _NPD // _NS


def _make_seg_sum(compute_deg):
    mesh = plsc.VectorSubcoreMesh(
        core_axis_name="c", subcore_axis_name="s",
        num_cores=_NC, num_subcores=_NS)

    out_type = [jax.ShapeDtypeStruct((_NC, _NP, _D), jnp.float32)]
    if compute_deg:
        out_type.append(jax.ShapeDtypeStruct((_NC * _NPD,), jnp.float32))

    @functools.partial(
        pl.kernel,
        out_type=tuple(out_type),
        mesh=mesh,
        scratch_types=[
            pltpu.VMEM((_BLK, _B), jnp.int32),
            pltpu.VMEM((_BLK, _B), jnp.int32),
            pltpu.VMEM((_NB, _B, _D), jnp.float32),
            pltpu.VMEM((_B,), jnp.float32),
            pltpu.VMEM_SHARED((_NP, _D), jnp.float32),
            pltpu.VMEM_SHARED((_NPD,), jnp.float32),
            pltpu.SemaphoreType.DMA,
            pltpu.SemaphoreType.DMA,
        ],
    )
    def seg_sum(table_hbm, edges_hbm, zeros2_hbm, zeros1_hbm, ones_hbm,
                *refs):
        if compute_deg:
            out_hbm, deg_hbm = refs[0], refs[1]
            rest = refs[2:]
        else:
            out_hbm = refs[0]
            rest = refs[1:]
        src_v, dst_v, rows_v, ones_v, acc_sh, deg_sh, gsem, dA = rest

        c = lax.axis_index("c")
        s = lax.axis_index("s")
        wid = s * _NC + c
        r0 = s * _RPT
        pltpu.sync_copy(zeros2_hbm, rows_v.at[0])
        for i in range(7):
            pltpu.sync_copy(rows_v.at[0], acc_sh.at[pl.ds(r0 + i * _B, _B)])
        pltpu.sync_copy(rows_v.at[0, pl.ds(0, _RPT - 7 * _B)],
                        acc_sh.at[pl.ds(r0 + 7 * _B, _RPT - 7 * _B)])
        if compute_deg:
            pltpu.sync_copy(zeros1_hbm, deg_sh.at[pl.ds(s * _RPD, _RPD)])
            pltpu.sync_copy(ones_hbm, ones_v)
        plsc.subcore_barrier()

        def fire_gather(j):
            pltpu.async_copy(table_hbm.at[src_v.at[j]],
                             rows_v.at[lax.rem(j, _NB)], gsem)

        def drain_gather():
            pltpu.make_async_copy(
                zeros2_hbm.at[pl.ds(0, _B)], rows_v.at[0], gsem).wait()

        def do_scatter(j):
            if compute_deg:
                d = pltpu.async_copy(ones_v, deg_sh.at[dst_v.at[j]], dA,
                                     add=True)
                pltpu.sync_copy(rows_v.at[lax.rem(j, _NB)],
                                acc_sh.at[dst_v.at[j]], add=True)
                d.wait()
            else:
                pltpu.sync_copy(rows_v.at[lax.rem(j, _NB)],
                                acc_sh.at[dst_v.at[j]], add=True)

        def block(k, carry):
            pltpu.sync_copy(edges_hbm.at[0, wid, k], src_v)
            pltpu.sync_copy(edges_hbm.at[1, wid, k], dst_v)
            for j in range(_LG):
                fire_gather(j)

            def body(j, c2):
                drain_gather()
                do_scatter(j)
                fire_gather(j + _LG)
                return c2

            lax.fori_loop(0, _BLK - _LG, body, 0)
            for j in range(_BLK - _LG, _BLK):
                drain_gather()
                do_scatter(j)
            return carry

        lax.fori_loop(0, _NBLK, block, 0)
        plsc.subcore_barrier()
        pltpu.sync_copy(acc_sh.at[pl.ds(r0, _RPT)],
                        out_hbm.at[c, pl.ds(r0, _RPT)])
        if compute_deg:
            pltpu.sync_copy(deg_sh.at[pl.ds(s * _RPD, _RPD)],
                            deg_hbm.at[pl.ds(c * _NPD + s * _RPD, _RPD)])

    return seg_sum


_seg_sum_l1 = _make_seg_sum(True)
_seg_sum_l2 = _make_seg_sum(False)


def _bn_relu_tail(h, g, b):
    m = jnp.mean(h, axis=0, keepdims=True)
    d = h - m
    v = jnp.mean(d * d, axis=0, keepdims=True)
    return jnp.maximum(g * d * lax.rsqrt(v + 1e-5) + b, 0.0)


def _tc_body(p_ref, x_ref, deg_ref, wl_ref, bl_ref,
             wr_ref, g_ref, b_ref, h_ref):
    degs = deg_ref[0] + deg_ref[1]
    dinv2 = 1.0 / jnp.maximum(degs[:_NP // _D], 1.0)
    psum = p_ref[0] + p_ref[1]
    scaled = psum.reshape(_NP // _D, _D, _D) * dinv2[:, :, None]
    agg = scaled.reshape(_NP, _D)[:_N]
    h = (jnp.dot(agg, wl_ref[...], preferred_element_type=jnp.float32)
         + bl_ref[...]
         + jnp.dot(x_ref[...], wr_ref[...], preferred_element_type=jnp.float32))
    h_ref[...] = _bn_relu_tail(h, g_ref[...], b_ref[...])


def _tc_layer(p, x, deg, Wl, bl, Wr, g, b):
    return pl.pallas_call(
        _tc_body,
        out_shape=jax.ShapeDtypeStruct((_N, _D), jnp.float32),
    )(p, x, deg, Wl, bl.reshape(1, _D), Wr,
      g.reshape(1, _D), b.reshape(1, _D))


def kernel(x, edge_index, Wl0, bl0, Wr0, g0, b0, Wl1, bl1, Wr1, g1, b1):
    edges = edge_index.reshape(2, _NW, _NBLK, _BLK, _B)
    zeros2 = jnp.zeros((_B, _D), jnp.float32)
    zeros1 = jnp.zeros((_RPD,), jnp.float32)
    ones = jnp.ones((_B,), jnp.float32)

    part1, deg = _seg_sum_l1(x, edges, zeros2, zeros1, ones)
    deg = deg.reshape(_NC, _NPD // _D, _D)

    h1 = _tc_layer(part1, x, deg, Wl0, bl0, Wr0, g0, b0)

    (part2,) = _seg_sum_l2(h1, edges, zeros2, zeros1, ones)

    h2 = _tc_layer(part2, h1, deg, Wl1, bl1, Wr1, g1, b1)

    return h2


# speedup vs baseline: 1.0973x; 1.0205x over previous
import functools

import jax
import jax.numpy as jnp
from jax import lax
from jax.experimental import pallas as pl
from jax.experimental.pallas import tpu as pltpu
from jax.experimental.pallas import tpu_sc as plsc

_N = 10000
_E = 320000
_D = 128
_NC = 2
_NS = 16
_NW = _NC * _NS
_EPT = _E // _NW
_B = 80
_CH = _EPT // _B
_BLK = 25
_NBLK = _CH // _BLK
_NB = 3
_LG = 3
_NP = 10112
_RPT = _NP // _NS
_NPD = 10240
_RPD = _NPD // _NS


def _make_seg_sum(compute_deg):
    mesh = plsc.VectorSubcoreMesh(
        core_axis_name="c", subcore_axis_name="s",
        num_cores=_NC, num_subcores=_NS)

    out_type = [jax.ShapeDtypeStruct((_NC, _NP, _D), jnp.float32)]
    if compute_deg:
        out_type.append(jax.ShapeDtypeStruct((_NC * _NPD,), jnp.float32))

    @functools.partial(
        pl.kernel,
        out_type=tuple(out_type),
        mesh=mesh,
        scratch_types=[
            pltpu.VMEM((2, _BLK, _B), jnp.int32),
            pltpu.VMEM((2, _BLK, _B), jnp.int32),
            pltpu.VMEM((_NB, _B, _D), jnp.float32),
            pltpu.VMEM((_B,), jnp.float32),
            pltpu.VMEM_SHARED((_NP, _D), jnp.float32),
            pltpu.VMEM_SHARED((_NPD,), jnp.float32),
            pltpu.SemaphoreType.DMA,
            pltpu.SemaphoreType.DMA,
            pltpu.SemaphoreType.DMA,
        ],
    )
    def seg_sum(table_hbm, edges_hbm, zeros2_hbm, zeros1_hbm, ones_hbm,
                *refs):
        if compute_deg:
            out_hbm, deg_hbm = refs[0], refs[1]
            rest = refs[2:]
        else:
            out_hbm = refs[0]
            rest = refs[1:]
        src_v, dst_v, rows_v, ones_v, acc_sh, deg_sh, gsem, dA, isem = rest

        c = lax.axis_index("c")
        s = lax.axis_index("s")
        wid = s * _NC + c
        r0 = s * _RPT
        pltpu.sync_copy(zeros2_hbm, rows_v.at[0])
        for i in range(7):
            pltpu.sync_copy(rows_v.at[0], acc_sh.at[pl.ds(r0 + i * _B, _B)])
        pltpu.sync_copy(rows_v.at[0, pl.ds(0, _RPT - 7 * _B)],
                        acc_sh.at[pl.ds(r0 + 7 * _B, _RPT - 7 * _B)])
        if compute_deg:
            pltpu.sync_copy(zeros1_hbm, deg_sh.at[pl.ds(s * _RPD, _RPD)])
            pltpu.sync_copy(ones_hbm, ones_v)
        plsc.subcore_barrier()

        def fire_gather(sl, j):
            pltpu.async_copy(table_hbm.at[src_v.at[sl, j]],
                             rows_v.at[lax.rem(j, _NB)], gsem)

        def drain_gather():
            pltpu.make_async_copy(
                zeros2_hbm.at[pl.ds(0, _B)], rows_v.at[0], gsem).wait()

        def do_scatter(sl, j):
            if compute_deg:
                d = pltpu.async_copy(ones_v, deg_sh.at[dst_v.at[sl, j]], dA,
                                     add=True)
                pltpu.sync_copy(rows_v.at[lax.rem(j, _NB)],
                                acc_sh.at[dst_v.at[sl, j]], add=True)
                d.wait()
            else:
                pltpu.sync_copy(rows_v.at[lax.rem(j, _NB)],
                                acc_sh.at[dst_v.at[sl, j]], add=True)

        pltpu.sync_copy(edges_hbm.at[0, wid, 0], src_v.at[0])
        pltpu.sync_copy(edges_hbm.at[1, wid, 0], dst_v.at[0])

        def block(k, carry):
            sl = lax.rem(k, 2)
            nsl = 1 - sl

            @pl.when(k + 1 < _NBLK)
            def _():
                pltpu.async_copy(edges_hbm.at[0, wid, k + 1],
                                 src_v.at[nsl], isem)
                pltpu.async_copy(edges_hbm.at[1, wid, k + 1],
                                 dst_v.at[nsl], isem)

            for j in range(_LG):
                fire_gather(sl, j)

            def body(j, c2):
                drain_gather()
                do_scatter(sl, j)
                fire_gather(sl, j + _LG)
                return c2

            lax.fori_loop(0, _BLK - _LG, body, 0)
            for j in range(_BLK - _LG, _BLK):
                drain_gather()
                do_scatter(sl, j)

            @pl.when(k + 1 < _NBLK)
            def _():
                pltpu.make_async_copy(edges_hbm.at[0, wid, 0],
                                      src_v.at[0], isem).wait()
                pltpu.make_async_copy(edges_hbm.at[1, wid, 0],
                                      dst_v.at[0], isem).wait()
            return carry

        lax.fori_loop(0, _NBLK, block, 0)
        plsc.subcore_barrier()
        pltpu.sync_copy(acc_sh.at[pl.ds(r0, _RPT)],
                        out_hbm.at[c, pl.ds(r0, _RPT)])
        if compute_deg:
            pltpu.sync_copy(deg_sh.at[pl.ds(s * _RPD, _RPD)],
                            deg_hbm.at[pl.ds(c * _NPD + s * _RPD, _RPD)])

    return seg_sum


_seg_sum_l1 = _make_seg_sum(True)
_seg_sum_l2 = _make_seg_sum(False)


def _bn_relu_tail(h, g, b):
    m = jnp.mean(h, axis=0, keepdims=True)
    d = h - m
    v = jnp.mean(d * d, axis=0, keepdims=True)
    return jnp.maximum(g * d * lax.rsqrt(v + 1e-5) + b, 0.0)


def _tc_body(p_ref, x_ref, deg_ref, wl_ref, bl_ref,
             wr_ref, g_ref, b_ref, h_ref):
    degs = deg_ref[0] + deg_ref[1]
    dinv2 = 1.0 / jnp.maximum(degs[:_NP // _D], 1.0)
    psum = p_ref[0] + p_ref[1]
    scaled = psum.reshape(_NP // _D, _D, _D) * dinv2[:, :, None]
    agg = scaled.reshape(_NP, _D)[:_N]
    h = (jnp.dot(agg, wl_ref[...], preferred_element_type=jnp.float32)
         + bl_ref[...]
         + jnp.dot(x_ref[...], wr_ref[...], preferred_element_type=jnp.float32))
    h_ref[...] = _bn_relu_tail(h, g_ref[...], b_ref[...])


def _tc_layer(p, x, deg, Wl, bl, Wr, g, b):
    return pl.pallas_call(
        _tc_body,
        out_shape=jax.ShapeDtypeStruct((_N, _D), jnp.float32),
    )(p, x, deg, Wl, bl.reshape(1, _D), Wr,
      g.reshape(1, _D), b.reshape(1, _D))


def kernel(x, edge_index, Wl0, bl0, Wr0, g0, b0, Wl1, bl1, Wr1, g1, b1):
    edges = edge_index.reshape(2, _NW, _NBLK, _BLK, _B)
    zeros2 = jnp.zeros((_B, _D), jnp.float32)
    zeros1 = jnp.zeros((_RPD,), jnp.float32)
    ones = jnp.ones((_B,), jnp.float32)

    part1, deg = _seg_sum_l1(x, edges, zeros2, zeros1, ones)
    deg = deg.reshape(_NC, _NPD // _D, _D)

    h1 = _tc_layer(part1, x, deg, Wl0, bl0, Wr0, g0, b0)

    (part2,) = _seg_sum_l2(h1, edges, zeros2, zeros1, ones)

    h2 = _tc_layer(part2, h1, deg, Wl1, bl1, Wr1, g1, b1)

    return h2
